# baseline (speedup 1.0000x reference)
import functools

import jax
import jax.numpy as jnp
from jax import lax
from jax.experimental import pallas as pl
from jax.experimental.pallas import tpu as pltpu
from jax.experimental.pallas import tpu_sc as plsc

N_NODES = 100000
N_EDGES = 3200000
N_GRAPHS = 256
N_PARTS = 2048

NC = 2
NS = 16
NW = NC * NS
LANES = 16

CHUNK = 3200
N_CHUNKS = N_EDGES // CHUNK
N_SLOTS = N_CHUNKS // NW
N_PAIRS = N_SLOTS // 2
N_EXTRA = N_CHUNKS - N_SLOTS * NW

_MESH = plsc.VectorSubcoreMesh(core_axis_name="c", subcore_axis_name="s")


def _build_table(pb_v, start_v, end_v, tbl_v):
    lane = lax.broadcasted_iota(jnp.int32, (LANES,), 0)
    for g in range(N_GRAPHS // LANES):
        s = pl.ds(g * LANES, LANES)
        start_v[s] = jnp.zeros((LANES,), jnp.int32)
        end_v[s] = jnp.full((LANES,), -1, jnp.int32)

    def scan_body(i, _):
        e_vec = i * LANES + lane
        cur = pb_v[pl.ds(i * LANES, LANES)]
        prev = plsc.load_gather(pb_v, [jnp.maximum(e_vec - 1, 0)])
        nxt = plsc.load_gather(pb_v, [jnp.minimum(e_vec + 1, N_PARTS - 1)])
        plsc.store_scatter(start_v, [cur], e_vec,
                           mask=(cur != prev) | (e_vec == 0))
        plsc.store_scatter(end_v, [cur], e_vec,
                           mask=(cur != nxt) | (e_vec == N_PARTS - 1))
        return 0

    lax.fori_loop(0, N_PARTS // LANES, scan_body, 0)

    for g in range(N_GRAPHS // LANES):
        s = pl.ds(g * LANES, LANES)
        d = (end_v[s] - start_v[s] + 1).astype(jnp.float32)
        i32v = plsc.bitcast(d, jnp.int32)
        y = plsc.bitcast(jnp.int32(0x5F3759DF) - (i32v >> 1), jnp.float32)
        hd = 0.5 * d
        y = y * (1.5 - hd * y * y)
        y = y * (1.5 - hd * y * y)
        y = y * (1.5 - hd * y * y)
        tbl_v[s] = jnp.where(d == 0.0, jnp.float32(jnp.inf), y)


@functools.partial(
    pl.kernel,
    out_type=jax.ShapeDtypeStruct((N_EDGES,), jnp.float32),
    mesh=_MESH,
    compiler_params=pltpu.CompilerParams(needs_layout_passes=False),
    scratch_types=[
        pltpu.VMEM((N_NODES,), jnp.int32),
        pltpu.VMEM((N_GRAPHS,), jnp.float32),
        pltpu.VMEM((N_PARTS,), jnp.int32),
        pltpu.VMEM((N_GRAPHS,), jnp.int32),
        pltpu.VMEM((N_GRAPHS,), jnp.int32),
        pltpu.VMEM((2, CHUNK), jnp.int32),
        pltpu.VMEM((2, CHUNK), jnp.int32),
        pltpu.VMEM((CHUNK,), jnp.float32),
        pltpu.VMEM((CHUNK,), jnp.float32),
        pltpu.VMEM((CHUNK,), jnp.float32),
        pltpu.VMEM((CHUNK,), jnp.float32),
        pltpu.SemaphoreType.DMA,
        pltpu.SemaphoreType.DMA,
        pltpu.SemaphoreType.DMA,
        pltpu.SemaphoreType.DMA,
        pltpu.SemaphoreType.DMA,
    ],
)
def _edge_kernel(ei_hbm, w_hbm, sb_hbm, pb_hbm, out_hbm,
                 sb_v, tbl_v, pb_v, start_v, end_v,
                 idx0, idx1, w0, w1, o0, o1,
                 isem0, isem1, osem0, osem1, ssem):
    sid = lax.axis_index("s")
    wid = sid * NC + lax.axis_index("c")

    bufs = ((idx0, w0, o0, isem0, osem0), (idx1, w1, o1, isem1, osem1))

    def issue_in(k, idx_v, w_v, isem):
        base = (k * NW + wid) * CHUNK
        pltpu.async_copy(ei_hbm.at[:, pl.ds(base, CHUNK)], idx_v, isem)
        pltpu.async_copy(w_hbm.at[pl.ds(base, CHUNK)], w_v, isem)

    def wait_in(idx_v, w_v, isem):
        pltpu.make_async_copy(ei_hbm.at[:, pl.ds(0, CHUNK)], idx_v, isem).wait()
        pltpu.make_async_copy(w_hbm.at[pl.ds(0, CHUNK)], w_v, isem).wait()

    def wait_out(o_v, osem):
        pltpu.make_async_copy(o_v, out_hbm.at[pl.ds(0, CHUNK)], osem).wait()

    def compute(idx_v, w_v, o_v):
        @plsc.parallel_loop(0, CHUNK, step=LANES, unroll=16)
        def _(e):
            s = pl.ds(e, LANES)
            gi = plsc.load_gather(sb_v, [idx_v[0, s]])
            v = plsc.load_gather(tbl_v, [gi])
            o_v[s] = v * w_v[s]

    issue_in(0, idx0, w0, isem0)
    issue_in(1, idx1, w1, isem1)

    pltpu.async_copy(sb_hbm, sb_v, ssem)
    pltpu.sync_copy(pb_hbm, pb_v)
    _build_table(pb_v, start_v, end_v, tbl_v)
    pltpu.make_async_copy(sb_hbm, sb_v, ssem).wait()

    def pair_body(p, _):
        for b in (0, 1):
            idx_v, w_v, o_v, isem, osem = bufs[b]
            k = p * 2 + b
            wait_in(idx_v, w_v, isem)

            @pl.when(p >= 1)
            def _():
                wait_out(o_v, osem)

            compute(idx_v, w_v, o_v)
            base = (k * NW + wid) * CHUNK
            pltpu.async_copy(o_v, out_hbm.at[pl.ds(base, CHUNK)], osem)

            if b == 0:
                issue_in(k + 2, idx_v, w_v, isem)
            else:
                @pl.when(p < N_PAIRS - 1)
                def _():
                    issue_in(k + 2, idx_v, w_v, isem)
        return 0

    lax.fori_loop(0, N_PAIRS, pair_body, 0)

    wait_in(idx0, w0, isem0)
    wait_out(o0, osem0)
    compute(idx0, w0, o0)
    tail_base = ((N_SLOTS - 1) * NW + wid) * CHUNK
    pltpu.async_copy(o0, out_hbm.at[pl.ds(tail_base, CHUNK)], osem0)
    wait_out(o1, osem1)

    @pl.when(wid < N_EXTRA)
    def _():
        base = (N_SLOTS * NW + wid) * CHUNK
        pltpu.sync_copy(ei_hbm.at[:, pl.ds(base, CHUNK)], idx1)
        pltpu.sync_copy(w_hbm.at[pl.ds(base, CHUNK)], w1)
        compute(idx1, w1, o1)
        pltpu.sync_copy(o1, out_hbm.at[pl.ds(base, CHUNK)])

    wait_out(o0, osem0)


def kernel(edge_index, surface_batch, part_batch, edge_weight):
    return _edge_kernel(edge_index, edge_weight, surface_batch, part_batch)


# speedup vs baseline: 1.0680x; 1.0680x over previous
import functools

import jax
import jax.numpy as jnp
from jax import lax
from jax.experimental import pallas as pl
from jax.experimental.pallas import tpu as pltpu
from jax.experimental.pallas import tpu_sc as plsc

N_NODES = 100000
N_EDGES = 3200000
N_GRAPHS = 256
N_PARTS = 2048

NC = 2
NS = 16
NW = NC * NS
LANES = 16

CHUNK = 2560
N_CHUNKS = N_EDGES // CHUNK
N_SLOTS = N_CHUNKS // NW
N_PAIRS = N_SLOTS // 2
N_EXTRA = N_CHUNKS - N_SLOTS * NW

_MESH = plsc.VectorSubcoreMesh(core_axis_name="c", subcore_axis_name="s")


def _build_table(pb_v, start_v, end_v, tbl_v):
    lane = lax.broadcasted_iota(jnp.int32, (LANES,), 0)
    for g in range(N_GRAPHS // LANES):
        s = pl.ds(g * LANES, LANES)
        start_v[s] = jnp.zeros((LANES,), jnp.int32)
        end_v[s] = jnp.full((LANES,), -1, jnp.int32)

    def scan_body(i, _):
        e_vec = i * LANES + lane
        cur = pb_v[pl.ds(i * LANES, LANES)]
        prev = plsc.load_gather(pb_v, [jnp.maximum(e_vec - 1, 0)])
        nxt = plsc.load_gather(pb_v, [jnp.minimum(e_vec + 1, N_PARTS - 1)])
        plsc.store_scatter(start_v, [cur], e_vec,
                           mask=(cur != prev) | (e_vec == 0))
        plsc.store_scatter(end_v, [cur], e_vec,
                           mask=(cur != nxt) | (e_vec == N_PARTS - 1))
        return 0

    lax.fori_loop(0, N_PARTS // LANES, scan_body, 0)

    for g in range(N_GRAPHS // LANES):
        s = pl.ds(g * LANES, LANES)
        d = (end_v[s] - start_v[s] + 1).astype(jnp.float32)
        i32v = plsc.bitcast(d, jnp.int32)
        y = plsc.bitcast(jnp.int32(0x5F3759DF) - (i32v >> 1), jnp.float32)
        hd = 0.5 * d
        y = y * (1.5 - hd * y * y)
        y = y * (1.5 - hd * y * y)
        y = y * (1.5 - hd * y * y)
        tbl_v[s] = jnp.where(d == 0.0, jnp.float32(jnp.inf), y)


@functools.partial(
    pl.kernel,
    out_type=jax.ShapeDtypeStruct((N_EDGES,), jnp.float32),
    mesh=_MESH,
    compiler_params=pltpu.CompilerParams(needs_layout_passes=False),
    scratch_types=[
        pltpu.VMEM_SHARED((N_NODES,), jnp.int32),
        pltpu.VMEM((N_NODES,), jnp.int32),
        pltpu.VMEM((N_GRAPHS,), jnp.float32),
        pltpu.VMEM((N_PARTS,), jnp.int32),
        pltpu.VMEM((N_GRAPHS,), jnp.int32),
        pltpu.VMEM((N_GRAPHS,), jnp.int32),
        pltpu.VMEM((2, CHUNK), jnp.int32),
        pltpu.VMEM((2, CHUNK), jnp.int32),
        pltpu.VMEM((CHUNK,), jnp.float32),
        pltpu.VMEM((CHUNK,), jnp.float32),
        pltpu.VMEM((CHUNK,), jnp.float32),
        pltpu.VMEM((CHUNK,), jnp.float32),
        pltpu.SemaphoreType.DMA,
        pltpu.SemaphoreType.DMA,
        pltpu.SemaphoreType.DMA,
        pltpu.SemaphoreType.DMA,
        pltpu.SemaphoreType.DMA,
    ],
)
def _edge_kernel(ei_hbm, w_hbm, sb_hbm, pb_hbm, out_hbm,
                 sb_sh, sb_v, tbl_v, pb_v, start_v, end_v,
                 idx0, idx1, w0, w1, o0, o1,
                 isem0, isem1, osem0, osem1, ssem):
    sid = lax.axis_index("s")
    wid = sid * NC + lax.axis_index("c")

    bufs = ((idx0, w0, o0, isem0, osem0), (idx1, w1, o1, isem1, osem1))

    def issue_in(k, idx_v, w_v, isem):
        base = (k * NW + wid) * CHUNK
        pltpu.async_copy(ei_hbm.at[:, pl.ds(base, CHUNK)], idx_v, isem)
        pltpu.async_copy(w_hbm.at[pl.ds(base, CHUNK)], w_v, isem)

    def wait_in(idx_v, w_v, isem):
        pltpu.make_async_copy(ei_hbm.at[:, pl.ds(0, CHUNK)], idx_v, isem).wait()
        pltpu.make_async_copy(w_hbm.at[pl.ds(0, CHUNK)], w_v, isem).wait()

    def wait_out(o_v, osem):
        pltpu.make_async_copy(o_v, out_hbm.at[pl.ds(0, CHUNK)], osem).wait()

    def compute(idx_v, w_v, o_v):
        @plsc.parallel_loop(0, CHUNK, step=LANES, unroll=16)
        def _(e):
            s = pl.ds(e, LANES)
            gi = plsc.load_gather(sb_v, [idx_v[0, s]])
            v = plsc.load_gather(tbl_v, [gi])
            o_v[s] = v * w_v[s]

    issue_in(0, idx0, w0, isem0)
    issue_in(1, idx1, w1, isem1)

    @pl.when(sid == 0)
    def _():
        pltpu.async_copy(sb_hbm, sb_sh, ssem)

    pltpu.sync_copy(pb_hbm, pb_v)
    _build_table(pb_v, start_v, end_v, tbl_v)

    @pl.when(sid == 0)
    def _():
        pltpu.make_async_copy(sb_hbm, sb_sh, ssem).wait()

    plsc.subcore_barrier()
    pltpu.sync_copy(sb_sh, sb_v)

    def pair_body(p, _):
        for b in (0, 1):
            idx_v, w_v, o_v, isem, osem = bufs[b]
            k = p * 2 + b
            wait_in(idx_v, w_v, isem)

            @pl.when(p >= 1)
            def _():
                wait_out(o_v, osem)

            compute(idx_v, w_v, o_v)
            base = (k * NW + wid) * CHUNK
            pltpu.async_copy(o_v, out_hbm.at[pl.ds(base, CHUNK)], osem)

            if b == 0:
                issue_in(k + 2, idx_v, w_v, isem)
            else:
                @pl.when(p < N_PAIRS - 1)
                def _():
                    issue_in(k + 2, idx_v, w_v, isem)
        return 0

    lax.fori_loop(0, N_PAIRS, pair_body, 0)

    wait_in(idx0, w0, isem0)
    wait_out(o0, osem0)
    compute(idx0, w0, o0)
    tail_base = ((N_SLOTS - 1) * NW + wid) * CHUNK
    pltpu.async_copy(o0, out_hbm.at[pl.ds(tail_base, CHUNK)], osem0)
    wait_out(o1, osem1)

    @pl.when(wid < N_EXTRA)
    def _():
        base = (N_SLOTS * NW + wid) * CHUNK
        pltpu.sync_copy(ei_hbm.at[:, pl.ds(base, CHUNK)], idx1)
        pltpu.sync_copy(w_hbm.at[pl.ds(base, CHUNK)], w1)
        compute(idx1, w1, o1)
        pltpu.sync_copy(o1, out_hbm.at[pl.ds(base, CHUNK)])

    wait_out(o0, osem0)


def kernel(edge_index, surface_batch, part_batch, edge_weight):
    return _edge_kernel(edge_index, edge_weight, surface_batch, part_batch)
